# baseline (speedup 1.0000x reference)
import functools

import jax
import jax.numpy as jnp
from jax import lax
from jax.experimental import pallas as pl
from jax.experimental.pallas import tpu as pltpu
from jax.experimental.pallas import tpu_sc as plsc

_EPS = 1e-06
_NC = 2
_NS = 16
_L = 16
_SLAB = 128

_LN2 = 0.6931471805599453


def _make_sc_kernel(n_nodes, batch, n_children):
    n_slabs = batch // _SLAB
    n_cores = 1
    mesh = plsc.VectorSubcoreMesh(
        core_axis_name="c", subcore_axis_name="s", num_cores=n_cores)

    @functools.partial(
        pl.kernel,
        out_type=jax.ShapeDtypeStruct((batch,), jnp.float32),
        mesh=mesh,
        scratch_types=[
            pltpu.VMEM((n_children,), jnp.int32),
            pltpu.VMEM((n_children, _SLAB), jnp.float32),
            pltpu.VMEM((n_children, _L), jnp.float32),
            pltpu.VMEM((_SLAB,), jnp.float32),
            pltpu.VMEM((_SLAB,), jnp.float32),
            pltpu.SemaphoreType.DMA,
            pltpu.SemaphoreType.DMA,
            pltpu.SemaphoreType.DMA,
            pltpu.SemaphoreType.DMA,
            pltpu.SemaphoreType.DMA,
        ],
    )
    def sc_kernel(table_hbm, idx_hbm, w_hbm, out_hbm,
                  idx_v, rows_v, w_v, out_v, m_v,
                  sem_g0, sem_g1, sem_g2, sem_g3, sem_w):
        wid = lax.axis_index("s") * n_cores + lax.axis_index("c")

        @pl.when(wid < n_slabs)
        def _():
            base = wid * _SLAB
            n_chunk = n_children // 4
            cw = pltpu.async_copy(w_hbm, w_v, sem_w)
            pltpu.sync_copy(idx_hbm, idx_v)
            copies = []
            for k, sem in enumerate((sem_g0, sem_g1, sem_g2, sem_g3)):
                iv = idx_v[pl.ds(k * n_chunk, n_chunk)]
                copies.append(pltpu.async_copy(
                    table_hbm.at[iv, pl.ds(base, _SLAB)],
                    rows_v.at[pl.ds(k * n_chunk, n_chunk), :], sem))
            cw.wait()

            for k in range(4):
                copies[k].wait()

                def chunk_max_body(g, carry, k=k):
                    csl = pl.ds(g * _L, _L)
                    j0 = k * n_chunk
                    m0 = rows_v[j0, csl]
                    m1 = rows_v[j0 + 1, csl]
                    m2 = rows_v[j0 + 2, csl]
                    m3 = rows_v[j0 + 3, csl]
                    for j in range(j0 + 4, j0 + n_chunk, 4):
                        m0 = jnp.maximum(m0, rows_v[j, csl])
                        m1 = jnp.maximum(m1, rows_v[j + 1, csl])
                        m2 = jnp.maximum(m2, rows_v[j + 2, csl])
                        m3 = jnp.maximum(m3, rows_v[j + 3, csl])
                    mk = jnp.maximum(jnp.maximum(m0, m1), jnp.maximum(m2, m3))
                    if k > 0:
                        mk = jnp.maximum(mk, m_v[csl])
                    m_v[csl] = mk
                    return carry

                lax.fori_loop(0, _SLAB // _L, chunk_max_body, 0)

            def group_body(g, carry):
                csl = pl.ds(g * _L, _L)
                m = m_v[csl]

                def sum_body(jb, ss):
                    j = jb * 4
                    return (ss[0] + w_v[j] * jnp.exp(rows_v[j, csl] - m),
                            ss[1] + w_v[j + 1] * jnp.exp(rows_v[j + 1, csl] - m),
                            ss[2] + w_v[j + 2] * jnp.exp(rows_v[j + 2, csl] - m),
                            ss[3] + w_v[j + 3] * jnp.exp(rows_v[j + 3, csl] - m))

                z = jnp.zeros((_L,), jnp.float32)
                s0, s1, s2, s3 = lax.fori_loop(
                    0, n_children // 4, sum_body, (z, z, z, z))
                s = (s0 + s1) + (s2 + s3)

                bits = lax.bitcast_convert_type(s, jnp.int32)
                y = (bits.astype(jnp.float32) * jnp.float32(_LN2 / (1 << 23))
                     - jnp.float32(127 * _LN2))
                for _ in range(3):
                    y = y + s * jnp.exp(-y) - jnp.float32(1.0)

                out_v[csl] = m + y
                return carry

            lax.fori_loop(0, _SLAB // _L, group_body, 0)

            pltpu.sync_copy(out_v, out_hbm.at[pl.ds(base, _SLAB)])

    return sc_kernel


def kernel(function_values, weights, children_indices):
    n_nodes, batch = function_values.shape
    n_children = weights.shape[0]
    assert batch % _SLAB == 0 and batch // _SLAB <= _NC * _NS
    assert n_children % 4 == 0

    idx = children_indices.astype(jnp.int32)
    w_eps = jnp.broadcast_to(
        (weights + jnp.float32(_EPS))[:, None], (n_children, _L))
    sc_kernel = _make_sc_kernel(n_nodes, batch, n_children)
    return sc_kernel(function_values, idx, w_eps)


# speedup vs baseline: 1.0094x; 1.0094x over previous
import functools

import jax
import jax.numpy as jnp
from jax import lax
from jax.experimental import pallas as pl
from jax.experimental.pallas import tpu as pltpu
from jax.experimental.pallas import tpu_sc as plsc

_EPS = 1e-06
_NC = 2
_NS = 16
_L = 16
_SLAB = 128

_LN2 = 0.6931471805599453


def _make_sc_kernel(n_nodes, batch, n_children):
    n_slabs = batch // _SLAB
    n_cores = 1
    mesh = plsc.VectorSubcoreMesh(
        core_axis_name="c", subcore_axis_name="s", num_cores=n_cores)

    @functools.partial(
        pl.kernel,
        out_type=jax.ShapeDtypeStruct((batch,), jnp.float32),
        mesh=mesh,
        scratch_types=[
            pltpu.VMEM((n_children,), jnp.int32),
            pltpu.VMEM((n_children, _SLAB), jnp.float32),
            pltpu.VMEM((n_children, _L), jnp.float32),
            pltpu.VMEM((_SLAB,), jnp.float32),
            pltpu.SemaphoreType.DMA,
            pltpu.SemaphoreType.DMA,
        ],
    )
    def sc_kernel(table_hbm, idx_hbm, w_hbm, out_hbm,
                  idx_v, rows_v, w_v, out_v, sem_g, sem_w):
        wid = lax.axis_index("s") * n_cores + lax.axis_index("c")

        @pl.when(wid < n_slabs)
        def _():
            base = wid * _SLAB
            cw = pltpu.async_copy(w_hbm, w_v, sem_w)
            pltpu.sync_copy(idx_hbm, idx_v)
            cg = pltpu.async_copy(
                table_hbm.at[idx_v, pl.ds(base, _SLAB)], rows_v, sem_g
            )
            cw.wait()
            cg.wait()

            def group_body(g, carry):
                csl = pl.ds(g * _L, _L)

                def max_body(jb, ms):
                    j = jb * 4
                    return (jnp.maximum(ms[0], rows_v[j, csl]),
                            jnp.maximum(ms[1], rows_v[j + 1, csl]),
                            jnp.maximum(ms[2], rows_v[j + 2, csl]),
                            jnp.maximum(ms[3], rows_v[j + 3, csl]))

                m0, m1, m2, m3 = lax.fori_loop(
                    1, n_children // 4, max_body,
                    (rows_v[0, csl], rows_v[1, csl],
                     rows_v[2, csl], rows_v[3, csl]))
                m = jnp.maximum(jnp.maximum(m0, m1), jnp.maximum(m2, m3))

                def sum_body(jb, ss):
                    j = jb * 4
                    return (ss[0] + w_v[j] * jnp.exp(rows_v[j, csl] - m),
                            ss[1] + w_v[j + 1] * jnp.exp(rows_v[j + 1, csl] - m),
                            ss[2] + w_v[j + 2] * jnp.exp(rows_v[j + 2, csl] - m),
                            ss[3] + w_v[j + 3] * jnp.exp(rows_v[j + 3, csl] - m))

                z = jnp.zeros((_L,), jnp.float32)
                s0, s1, s2, s3 = lax.fori_loop(
                    0, n_children // 4, sum_body, (z, z, z, z))
                s = (s0 + s1) + (s2 + s3)

                bits = lax.bitcast_convert_type(s, jnp.int32)
                y = (bits.astype(jnp.float32) * jnp.float32(_LN2 / (1 << 23))
                     - jnp.float32(127 * _LN2))
                for _ in range(3):
                    y = y + s * jnp.exp(-y) - jnp.float32(1.0)

                out_v[csl] = m + y
                return carry

            lax.fori_loop(0, _SLAB // _L, group_body, 0)

            pltpu.sync_copy(out_v, out_hbm.at[pl.ds(base, _SLAB)])

    return sc_kernel


def kernel(function_values, weights, children_indices):
    n_nodes, batch = function_values.shape
    n_children = weights.shape[0]
    assert batch % _SLAB == 0 and batch // _SLAB <= _NC * _NS
    assert n_children % 4 == 0

    idx = children_indices.astype(jnp.int32)
    w_eps = jnp.broadcast_to(
        (weights + jnp.float32(_EPS))[:, None], (n_children, _L))
    sc_kernel = _make_sc_kernel(n_nodes, batch, n_children)
    return sc_kernel(function_values, idx, w_eps)
